# Initial kernel scaffold; baseline (speedup 1.0000x reference)
import jax
import jax.numpy as jnp
from jax.experimental import pallas as pl


def kernel(x, y):
    raise NotImplementedError("write your pallas kernel here")


# speedup vs baseline: 2.5509x; 2.5509x over previous
import functools
import math

import jax
import jax.numpy as jnp
from jax import lax
from jax.experimental import pallas as pl
from jax.experimental.pallas import tpu as pltpu
from jax.experimental.pallas import tpu_sc as plsc

N = 4096
V = 32000
PAD = 0
SMOOTH = 0.1
CONF = 1.0 - SMOOTH
EPS = SMOOTH / (V - 2)
ROW_CONST = CONF * math.log(CONF) + (V - 2) * EPS * math.log(EPS)

NC = 2
NS = 16
L = 16
NW = NC * NS
BPW = N // NW
RPR = V // L

BR = 512
BC = 3200
NRB = N // BR
NCB = V // BC


def _sc_gather_partials(x1, y):
    mesh = plsc.VectorSubcoreMesh(core_axis_name="c", subcore_axis_name="s")

    @functools.partial(
        pl.kernel,
        mesh=mesh,
        out_type=[
            jax.ShapeDtypeStruct((NW, L), jnp.float32),
            jax.ShapeDtypeStruct((NW, L), jnp.float32),
        ],
        scratch_types=[
            pltpu.VMEM((BPW,), jnp.int32),
            pltpu.VMEM((BPW,), jnp.int32),
            pltpu.VMEM((BPW,), jnp.float32),
            pltpu.VMEM((L,), jnp.float32),
            pltpu.VMEM((L,), jnp.float32),
            pltpu.SemaphoreType.DMA,
        ],
    )
    def k(x1_hbm, y_hbm, g_hbm, n_hbm, y_v, idx_v, vals_v, g_v, n_v, sem):
        wid = lax.axis_index("s") * NC + lax.axis_index("c")
        base = wid * BPW
        pltpu.sync_copy(y_hbm.at[pl.ds(base, BPW)], y_v)
        iv = lax.iota(jnp.int32, L)
        for j in range(BPW // L):
            yv = y_v[pl.ds(j * L, L)]
            row = base + j * L + iv
            idx_v[pl.ds(j * L, L)] = row * V + yv
        pltpu.async_copy(x1_hbm.at[idx_v], vals_v, sem).wait()
        acc = jnp.zeros((L,), jnp.float32)
        cnt = jnp.zeros((L,), jnp.float32)
        for j in range(BPW // L):
            yv = y_v[pl.ds(j * L, L)]
            g = vals_v[pl.ds(j * L, L)]
            m = yv != PAD
            acc = acc + jnp.where(m, g, 0.0)
            cnt = cnt + jnp.where(m, 1.0, 0.0)
        g_v[...] = acc
        n_v[...] = cnt
        pltpu.sync_copy(g_v, g_hbm.at[wid])
        pltpu.sync_copy(n_v, n_hbm.at[wid])

    return k(x1, y)


def _tc_body(y_ref, x_ref, out_ref, acc_ref, col0_ref):
    c = pl.program_id(1)
    blk = x_ref[...]
    psum = jnp.sum(blk, axis=1, keepdims=True)

    @pl.when(c == 0)
    def _():
        acc_ref[...] = psum
        col0_ref[...] = blk[:, 0:1]

    @pl.when(c != 0)
    def _():
        acc_ref[...] = acc_ref[...] + psum

    @pl.when(c == NCB - 1)
    def _():
        yv = y_ref[0]
        mask = yv != PAD
        diff = jnp.where(mask, col0_ref[...] - acc_ref[...], 0.0)
        val = jnp.sum(diff)
        r = pl.program_id(0)

        @pl.when(r == 0)
        def _():
            out_ref[0, 0] = val

        @pl.when(r != 0)
        def _():
            out_ref[0, 0] = out_ref[0, 0] + val


def _tc_masked_colsum(x, y3):
    return pl.pallas_call(
        _tc_body,
        grid=(NRB, NCB),
        in_specs=[
            pl.BlockSpec((1, BR, 1), lambda r, c: (r, 0, 0)),
            pl.BlockSpec((BR, BC), lambda r, c: (r, c)),
        ],
        out_specs=pl.BlockSpec(memory_space=pltpu.SMEM),
        out_shape=jax.ShapeDtypeStruct((1, 1), jnp.float32),
        scratch_shapes=[
            pltpu.VMEM((BR, 1), jnp.float32),
            pltpu.VMEM((BR, 1), jnp.float32),
        ],
    )(y3, x)


def kernel(x, y):
    g_part, n_part = _sc_gather_partials(x.reshape(N * V), y)
    b = _tc_masked_colsum(x, y.reshape(NRB, BR, 1))
    a = jnp.sum(g_part)
    k = jnp.sum(n_part)
    return (k * jnp.float32(ROW_CONST)
            - jnp.float32(CONF - EPS) * a
            + jnp.float32(EPS) * b[0, 0])
